# baseline (speedup 1.0000x reference)
import functools
import math

import jax
import jax.numpy as jnp
from jax import lax
from jax.experimental import pallas as pl
from jax.experimental.pallas import tpu as pltpu
from jax.experimental.pallas import tpu_sc as plsc

_LANES = 16
_NG = 4
_NS = 2


def _emb_body(b_per_w, chunk, nchunks, d,
              table_hbm, idx_hbm, out_hbm,
              idx_v, gbufs, sbufs, gsems, ssems):
    scale = jnp.float32(math.sqrt(d))
    wid = lax.axis_index("s") * 2 + lax.axis_index("c")
    base = wid * b_per_w
    pltpu.sync_copy(idx_hbm.at[pl.ds(base, b_per_w)], idx_v)

    def start_gather(g, b):
        off = pl.multiple_of(g * chunk, chunk)
        pltpu.async_copy(
            table_hbm.at[idx_v.at[pl.ds(off, chunk)]], gbufs[b], gsems[b]
        )

    def wait_gather(b):
        pltpu.make_async_copy(
            table_hbm.at[idx_v.at[pl.ds(0, chunk)]], gbufs[b], gsems[b]
        ).wait()

    def start_write(g, b):
        off = pl.multiple_of(g * chunk, chunk)
        pltpu.async_copy(
            sbufs[b], out_hbm.at[pl.ds(base + off, chunk)], ssems[b]
        )

    def wait_write(b):
        pltpu.make_async_copy(
            sbufs[b], out_hbm.at[pl.ds(base, chunk)], ssems[b]
        ).wait()

    def do_scale(gb, sb):
        gbuf, sbuf = gbufs[gb], sbufs[sb]

        def row_body(i, c):
            for j in range(d // _LANES):
                sl = pl.ds(j * _LANES, _LANES)
                sbuf[i, sl] = gbuf[i, sl] * scale
            return c

        lax.fori_loop(0, chunk, row_body, 0, unroll=4)

    for b in range(_NG):
        start_gather(b, b)

    for g in range(_NG):
        gb, sb = g % _NG, g % _NS
        wait_gather(gb)
        if g >= _NS:
            wait_write(sb)
        do_scale(gb, sb)
        start_gather(g + _NG, gb)
        start_write(g, sb)

    def round_body(r, carry):
        g0 = r * _NG
        for b in range(_NG):
            g = g0 + b
            sb = b % _NS
            wait_gather(b)
            wait_write(sb)
            do_scale(b, sb)
            start_gather(g + _NG, b)
            start_write(g, sb)
        return carry

    lax.fori_loop(1, nchunks // _NG - 1, round_body, 0)

    for b in range(_NG):
        g = nchunks - _NG + b
        sb = g % _NS
        wait_gather(b)
        wait_write(sb)
        do_scale(b, sb)
        start_write(g, sb)
    for sb in range(_NS):
        wait_write(sb)


def kernel(tokens, table):
    v, d = table.shape
    idx = tokens.reshape(-1).astype(jnp.int32)
    b = idx.shape[0]
    nw = 32
    b_per_w = b // nw
    chunk = 128
    nchunks = b_per_w // chunk

    mesh = plsc.VectorSubcoreMesh(core_axis_name="c", subcore_axis_name="s")
    f = pl.kernel(
        functools.partial(_emb_body, b_per_w, chunk, nchunks, d),
        mesh=mesh,
        compiler_params=pltpu.CompilerParams(use_tc_tiling_on_sc=False),
        out_type=jax.ShapeDtypeStruct((b, d), jnp.float32),
        scratch_types=[
            pltpu.VMEM((b_per_w,), jnp.int32),
            [pltpu.VMEM((chunk, d), jnp.float32) for _ in range(_NG)],
            [pltpu.VMEM((chunk, d), jnp.float32) for _ in range(_NS)],
            [pltpu.SemaphoreType.DMA for _ in range(_NG)],
            [pltpu.SemaphoreType.DMA for _ in range(_NS)],
        ],
    )
    out = f(table, idx)
    return out.reshape(*tokens.shape, d)
